# Initial kernel scaffold; baseline (speedup 1.0000x reference)
import functools

import jax
import jax.numpy as jnp
from jax import lax
from jax.experimental import pallas as pl
from jax.experimental.pallas import tpu as pltpu
from jax.experimental.pallas import tpu_sc as plsc

N = 10000
D = 128
E = 320000

NC = 2
NS = 16
NW = NC * NS

B = 128
NCH = 80
EPW = NCH * B
E_PAD = NW * EPW
NPAD = 10240
ROWS_PS = NPAD // NS
GARBAGE = N + 100
NBUF = 4

_mesh = plsc.VectorSubcoreMesh(core_axis_name="c", subcore_axis_name="s")


def _deg_body(dsts_hbm, ones_hbm, zrow_hbm, degp_hbm, dst_v, ones_v, zbuf,
              sem):
    cid = lax.axis_index("c")
    sid = lax.axis_index("s")
    wid = cid * NS + sid

    pltpu.sync_copy(ones_hbm, ones_v)
    pltpu.sync_copy(zrow_hbm, zbuf)
    pltpu.sync_copy(dsts_hbm.at[wid], dst_v)

    def run(acc):
        lo = sid * ROWS_PS
        pltpu.sync_copy(zbuf, acc.at[pl.ds(lo, ROWS_PS)])
        plsc.subcore_barrier()

        def chunk(j, _):
            pltpu.sync_copy(ones_v, acc.at[dst_v.at[j]], add=True)
            return 0

        lax.fori_loop(0, NCH, chunk, 0)
        plsc.subcore_barrier()
        pltpu.sync_copy(acc.at[pl.ds(lo, ROWS_PS)], zbuf)
        pltpu.sync_copy(zbuf, degp_hbm.at[cid, pl.ds(lo, ROWS_PS)])

    pl.run_scoped(run, pltpu.VMEM_SHARED((NPAD,), jnp.float32))


_deg_kernel = functools.partial(
    pl.kernel,
    out_type=jax.ShapeDtypeStruct((NC, NPAD), jnp.float32),
    mesh=_mesh,
    scratch_types=[
        pltpu.VMEM((NCH, B), jnp.int32),
        pltpu.VMEM((B,), jnp.float32),
        pltpu.VMEM((ROWS_PS,), jnp.float32),
        pltpu.SemaphoreType.DMA,
    ],
)(_deg_body)


def _make_agg(dfeat):
    def body(y_hbm, srcs_hbm, dsts_hbm, zb_hbm, out_hbm, src_v, dst_v, bufs,
             sem):
        cid = lax.axis_index("c")
        sid = lax.axis_index("s")
        wid = cid * NS + sid

        pltpu.sync_copy(srcs_hbm.at[wid], src_v)
        pltpu.sync_copy(dsts_hbm.at[wid], dst_v)
        pltpu.sync_copy(zb_hbm, bufs.at[0])

        def run(acc):
            lo = sid * ROWS_PS
            for k in range(ROWS_PS // B):
                pltpu.sync_copy(bufs.at[0], acc.at[pl.ds(lo + k * B, B)])
            plsc.subcore_barrier()

            def chunk4(i, _):
                j0 = i * NBUF
                hs = []
                for k in range(NBUF):
                    hs.append(pltpu.async_copy(
                        y_hbm.at[src_v.at[j0 + k]], bufs.at[k], sem))
                for k in range(NBUF):
                    hs[k].wait()
                    pltpu.sync_copy(bufs.at[k], acc.at[dst_v.at[j0 + k]],
                                    add=True)
                return 0

            lax.fori_loop(0, NCH // NBUF, chunk4, 0)
            plsc.subcore_barrier()
            for k in range(ROWS_PS // B):
                pltpu.sync_copy(acc.at[pl.ds(lo + k * B, B)], bufs.at[0])
                pltpu.sync_copy(bufs.at[0],
                                out_hbm.at[cid, pl.ds(lo + k * B, B)])

        pl.run_scoped(run, pltpu.VMEM_SHARED((NPAD, dfeat), jnp.float32))

    return functools.partial(
        pl.kernel,
        out_type=jax.ShapeDtypeStruct((NC, NPAD, dfeat), jnp.float32),
        mesh=_mesh,
        scratch_types=[
            pltpu.VMEM((NCH, B), jnp.int32),
            pltpu.VMEM((NCH, B), jnp.int32),
            pltpu.VMEM((NBUF, B, dfeat), jnp.float32),
            pltpu.SemaphoreType.DMA,
        ],
    )(body)


_agg_d = _make_agg(D)
_agg_2 = _make_agg(2)


def _tc_a_body(x_ref, w_ref, d0_ref, d1_ref, y_ref):
    deg = d0_ref[0, 0, :] + d1_ref[0, 0, :] + 1.0
    dis = lax.rsqrt(deg)
    xw = jnp.dot(x_ref[...], w_ref[...], preferred_element_type=jnp.float32)
    y_ref[...] = xw * dis[:, None]


def _tc_b_body(p0_ref, p1_ref, y1_ref, d0_ref, d1_ref, b1_ref, w2_ref,
               y2_ref):
    deg = d0_ref[0, 0, :] + d1_ref[0, 0, :] + 1.0
    dis = lax.rsqrt(deg)
    h = dis[:, None] * (p0_ref[...] + p1_ref[...] + y1_ref[...]) + b1_ref[...]
    h = jnp.maximum(h, 0.0)
    hw = jnp.dot(h, w2_ref[...], preferred_element_type=jnp.float32)
    y2_ref[...] = hw * dis[:, None]


def _tc_d_body(q0_ref, q1_ref, y2_ref, dp_ref, b2_ref, out_ref):
    deg = dp_ref[0, :] + dp_ref[1, :] + 1.0
    dis = lax.rsqrt(deg)
    out_ref[...] = dis[:, None] * (q0_ref[...] + q1_ref[...] + y2_ref[...]) \
        + b2_ref[...]


_RB = 1024
_G = NPAD // _RB


def _tc_a(xp, W1, d0, d1):
    return pl.pallas_call(
        _tc_a_body,
        grid=(_G,),
        in_specs=[
            pl.BlockSpec((_RB, D), lambda i: (i, 0)),
            pl.BlockSpec((D, D), lambda i: (0, 0)),
            pl.BlockSpec((1, 1, _RB), lambda i: (i, 0, 0)),
            pl.BlockSpec((1, 1, _RB), lambda i: (i, 0, 0)),
        ],
        out_specs=pl.BlockSpec((_RB, D), lambda i: (i, 0)),
        out_shape=jax.ShapeDtypeStruct((NPAD, D), jnp.float32),
    )(xp, W1, d0, d1)


def _tc_b(p0, p1, y1, d0, d1, b1, W2):
    return pl.pallas_call(
        _tc_b_body,
        grid=(_G,),
        in_specs=[
            pl.BlockSpec((_RB, D), lambda i: (i, 0)),
            pl.BlockSpec((_RB, D), lambda i: (i, 0)),
            pl.BlockSpec((_RB, D), lambda i: (i, 0)),
            pl.BlockSpec((1, 1, _RB), lambda i: (i, 0, 0)),
            pl.BlockSpec((1, 1, _RB), lambda i: (i, 0, 0)),
            pl.BlockSpec((1, D), lambda i: (0, 0)),
            pl.BlockSpec((D, 2), lambda i: (0, 0)),
        ],
        out_specs=pl.BlockSpec((_RB, 2), lambda i: (i, 0)),
        out_shape=jax.ShapeDtypeStruct((NPAD, 2), jnp.float32),
    )(p0, p1, y1, d0, d1, b1, W2)


def _tc_d(q0, q1, y2, degp, b2):
    return pl.pallas_call(
        _tc_d_body,
        in_specs=[
            pl.BlockSpec((NPAD, 2), lambda: (0, 0)),
            pl.BlockSpec((NPAD, 2), lambda: (0, 0)),
            pl.BlockSpec((NPAD, 2), lambda: (0, 0)),
            pl.BlockSpec((NC, NPAD), lambda: (0, 0)),
            pl.BlockSpec((1, 2), lambda: (0, 0)),
        ],
        out_specs=pl.BlockSpec((NPAD, 2), lambda: (0, 0)),
        out_shape=jax.ShapeDtypeStruct((NPAD, 2), jnp.float32),
    )(q0, q1, y2, degp, b2)


@jax.jit
def kernel(x, edge_index, W1, b1, W2, b2):
    ei = edge_index.astype(jnp.int32)
    src = jnp.concatenate(
        [ei[0], jnp.zeros((E_PAD - E,), jnp.int32)]).reshape(NW, NCH, B)
    dst = jnp.concatenate(
        [ei[1], jnp.full((E_PAD - E,), GARBAGE, jnp.int32)]).reshape(
            NW, NCH, B)
    xp = jnp.pad(x, ((0, NPAD - N), (0, 0)))

    ones = jnp.ones((B,), jnp.float32)
    zrow = jnp.zeros((ROWS_PS,), jnp.float32)
    zb_d = jnp.zeros((B, D), jnp.float32)
    zb_2 = jnp.zeros((B, 2), jnp.float32)

    degp = _deg_kernel(dst, ones, zrow)
    d0 = degp[0].reshape(_G, 1, _RB)
    d1 = degp[1].reshape(_G, 1, _RB)

    y1 = _tc_a(xp, W1, d0, d1)
    p = _agg_d(y1, src, dst, zb_d)
    y2 = _tc_b(p[0], p[1], y1, d0, d1, b1.reshape(1, D), W2)
    q = _agg_2(y2, src, dst, zb_2)
    out = _tc_d(q[0], q[1], y2, degp, b2.reshape(1, 2))
    return out[:N]


# speedup vs baseline: 5.9005x; 5.9005x over previous
import functools

import jax
import jax.numpy as jnp
from jax import lax
from jax.experimental import pallas as pl
from jax.experimental.pallas import tpu as pltpu
from jax.experimental.pallas import tpu_sc as plsc

N = 10000
D = 128
E = 320000

NC = 2
NS = 16
NW = NC * NS

B = 128
NCH = 80
E_PAD = NW * NCH * B
NPAD = 10240
GARBAGE = N + 100
NPASS = 3
RPP = 3456
ACC1_ROWS = RPP + 8
GARB_LOCAL = RPP
RPT1 = RPP // NS
OUT1_ROWS = NPASS * RPP
NBUF = 4

_mesh = plsc.VectorSubcoreMesh(core_axis_name="c", subcore_axis_name="s")


def _deg_body(dsts_hbm, degp_hbm, dst_v, acc, sem):
    cid = lax.axis_index("c")
    sid = lax.axis_index("s")
    wid = cid * NS + sid

    pltpu.sync_copy(dsts_hbm.at[wid], dst_v)

    def fill(i, _):
        acc[pl.ds(i * 16, 16)] = jnp.zeros((16,), jnp.float32)
        return 0

    lax.fori_loop(0, NPAD // 16, fill, 0)

    ones16 = jnp.ones((16,), jnp.float32)

    def chunk(j, _):
        for s in range(B // 16):
            idx = dst_v[j, pl.ds(s * 16, 16)]
            plsc.addupdate_scatter(acc, [idx], ones16)
        return 0

    lax.fori_loop(0, NCH, chunk, 0)
    pltpu.sync_copy(acc, degp_hbm.at[wid])


_deg_kernel = functools.partial(
    pl.kernel,
    out_type=jax.ShapeDtypeStruct((NW, NPAD), jnp.float32),
    mesh=_mesh,
    compiler_params=pltpu.CompilerParams(needs_layout_passes=False, internal_scratch_in_bytes=0),
    scratch_types=[
        pltpu.VMEM((NCH, B), jnp.int32),
        pltpu.VMEM((NPAD,), jnp.float32),
        pltpu.SemaphoreType.DMA,
    ],
)(_deg_body)


def _agg1_body(y_hbm, srcs_hbm, d0_hbm, d1_hbm, d2_hbm, out_hbm, src_v,
               dst_v, bufs, zbuf, acc, sem):
    cid = lax.axis_index("c")
    sid = lax.axis_index("s")
    wid = cid * NS + sid

    pltpu.sync_copy(srcs_hbm.at[wid], src_v)

    def zfill(i, _):
        zbuf[lax.div(i, 8), pl.ds(lax.rem(i, 8) * 16, 16)] = (
            jnp.zeros((16,), jnp.float32))
        return 0

    lax.fori_loop(0, B * 8, zfill, 0)

    lo = sid * RPT1
    for p, d_hbm in ((0, d0_hbm), (1, d1_hbm), (2, d2_hbm)):
        pltpu.sync_copy(d_hbm.at[wid], dst_v)
        for k, sz in ((0, B), (1, RPT1 - B)):
            pltpu.sync_copy(zbuf.at[pl.ds(0, sz)],
                            acc.at[pl.ds(lo + k * B, sz)])
        plsc.subcore_barrier()

        def chunk4(i, _):
            j0 = i * NBUF
            hs = []
            for k in range(NBUF):
                hs.append(pltpu.async_copy(
                    y_hbm.at[src_v.at[j0 + k]], bufs.at[k], sem))
            for k in range(NBUF):
                hs[k].wait()
                pltpu.sync_copy(bufs.at[k], acc.at[dst_v.at[j0 + k]],
                                add=True)
            return 0

        lax.fori_loop(0, NCH // NBUF, chunk4, 0)
        plsc.subcore_barrier()
        for k, sz in ((0, B), (1, RPT1 - B)):
            pltpu.sync_copy(acc.at[pl.ds(lo + k * B, sz)],
                            bufs.at[0].at[pl.ds(0, sz)])
            pltpu.sync_copy(
                bufs.at[0].at[pl.ds(0, sz)],
                out_hbm.at[cid, pl.ds(p * RPP + lo + k * B, sz)])
        plsc.subcore_barrier()


_agg1_kernel = functools.partial(
    pl.kernel,
    out_type=jax.ShapeDtypeStruct((NC, OUT1_ROWS, D), jnp.float32),
    mesh=_mesh,
    compiler_params=pltpu.CompilerParams(internal_scratch_in_bytes=0),
    scratch_types=[
        pltpu.VMEM((NCH, B), jnp.int32),
        pltpu.VMEM((NCH, B), jnp.int32),
        pltpu.VMEM((NBUF, B, D), jnp.float32),
        pltpu.VMEM((B, D), jnp.float32),
        pltpu.VMEM_SHARED((ACC1_ROWS, D), jnp.float32),
        pltpu.SemaphoreType.DMA,
    ],
)(_agg1_body)


def _agg2_body(yt_hbm, srcs_hbm, dsts_hbm, out_hbm, src_v, dst_v, bufs, acc,
               sem):
    cid = lax.axis_index("c")
    sid = lax.axis_index("s")
    wid = cid * NS + sid

    pltpu.sync_copy(srcs_hbm.at[wid], src_v)
    pltpu.sync_copy(dsts_hbm.at[wid], dst_v)

    def zfill(i, _):
        for f in range(2):
            acc[f, pl.ds(i * 16, 16)] = jnp.zeros((16,), jnp.float32)
        return 0

    lax.fori_loop(0, NPAD // 16, zfill, 0)

    for f in range(2):
        yf = yt_hbm.at[pl.ds(f * NPAD, NPAD)]
        fidx = jnp.full((16,), f, jnp.int32)

        def chunk4(i, _):
            j0 = i * NBUF
            hs = []
            for k in range(NBUF):
                hs.append(pltpu.async_copy(
                    yf.at[src_v.at[j0 + k]], bufs.at[k], sem))
            for k in range(NBUF):
                hs[k].wait()
                for s in range(B // 16):
                    vals = bufs[k, pl.ds(s * 16, 16)]
                    idx = dst_v[j0 + k, pl.ds(s * 16, 16)]
                    plsc.addupdate_scatter(acc, [fidx, idx], vals)
            return 0

        lax.fori_loop(0, NCH // NBUF, chunk4, 0)
    pltpu.sync_copy(acc, out_hbm.at[wid])


_agg2_kernel = functools.partial(
    pl.kernel,
    out_type=jax.ShapeDtypeStruct((NW, 2, NPAD), jnp.float32),
    mesh=_mesh,
    compiler_params=pltpu.CompilerParams(needs_layout_passes=False, internal_scratch_in_bytes=0),
    scratch_types=[
        pltpu.VMEM((NCH, B), jnp.int32),
        pltpu.VMEM((NCH, B), jnp.int32),
        pltpu.VMEM((NBUF, B), jnp.float32),
        pltpu.VMEM((2, NPAD), jnp.float32),
        pltpu.SemaphoreType.DMA,
    ],
)(_agg2_body)


def _tc_a_body(x_ref, w_ref, dp_ref, y_ref):
    deg = jnp.sum(dp_ref[0], axis=0) + 1.0
    dis = lax.rsqrt(deg)
    xw = jnp.dot(x_ref[...], w_ref[...], preferred_element_type=jnp.float32)
    y_ref[...] = xw * dis[:, None]


def _tc_b_body(p_ref, y1_ref, dp_ref, b1_ref, w2_ref, y2t_ref):
    deg = jnp.sum(dp_ref[0], axis=0) + 1.0
    dis = lax.rsqrt(deg)
    agg = p_ref[0] + p_ref[1]
    h = dis[:, None] * (agg + y1_ref[...]) + b1_ref[...]
    h = jnp.maximum(h, 0.0)
    hw = jnp.dot(h, w2_ref[...], preferred_element_type=jnp.float32)
    y2t_ref[...] = (hw * dis[:, None]).T


def _tc_d_body(q_ref, y2t_ref, dp_ref, b2_ref, out_ref):
    deg = jnp.sum(dp_ref[...], axis=0) + 1.0
    dis = lax.rsqrt(deg)
    agg = jnp.sum(q_ref[...], axis=0)
    out_ref[...] = dis[None, :] * (agg + y2t_ref[...]) + b2_ref[...]


_RB = 1024
_G = NPAD // _RB


def _tc_a(xp, W1, dp):
    return pl.pallas_call(
        _tc_a_body,
        grid=(_G,),
        in_specs=[
            pl.BlockSpec((_RB, D), lambda i: (i, 0)),
            pl.BlockSpec((D, D), lambda i: (0, 0)),
            pl.BlockSpec((1, NW, _RB), lambda i: (i, 0, 0)),
        ],
        out_specs=pl.BlockSpec((_RB, D), lambda i: (i, 0)),
        out_shape=jax.ShapeDtypeStruct((NPAD, D), jnp.float32),
    )(xp, W1, dp)


def _tc_b(p, y1, dp, b1, W2):
    return pl.pallas_call(
        _tc_b_body,
        grid=(_G,),
        in_specs=[
            pl.BlockSpec((NC, _RB, D), lambda i: (0, i, 0)),
            pl.BlockSpec((_RB, D), lambda i: (i, 0)),
            pl.BlockSpec((1, NW, _RB), lambda i: (i, 0, 0)),
            pl.BlockSpec((1, D), lambda i: (0, 0)),
            pl.BlockSpec((D, 2), lambda i: (0, 0)),
        ],
        out_specs=pl.BlockSpec((2, _RB), lambda i: (0, i)),
        out_shape=jax.ShapeDtypeStruct((2, NPAD), jnp.float32),
    )(p, y1, dp, b1, W2)


def _tc_d(q, y2t, dp, b2):
    return pl.pallas_call(
        _tc_d_body,
        in_specs=[
            pl.BlockSpec((NW, 2, NPAD), lambda: (0, 0, 0)),
            pl.BlockSpec((2, NPAD), lambda: (0, 0)),
            pl.BlockSpec((NW, NPAD), lambda: (0, 0)),
            pl.BlockSpec((2, 1), lambda: (0, 0)),
        ],
        out_specs=pl.BlockSpec((2, NPAD), lambda: (0, 0)),
        out_shape=jax.ShapeDtypeStruct((2, NPAD), jnp.float32),
    )(q, y2t, dp, b2)


@jax.jit
def kernel(x, edge_index, W1, b1, W2, b2):
    ei = edge_index.astype(jnp.int32)
    srcf = jnp.concatenate([ei[0], jnp.zeros((E_PAD - E,), jnp.int32)])
    dstf = jnp.concatenate([ei[1], jnp.full((E_PAD - E,), GARBAGE, jnp.int32)])
    src = srcf.reshape(NW, NCH, B)
    dst = dstf.reshape(NW, NCH, B)
    dst0 = jnp.where(dstf < RPP, dstf, GARB_LOCAL).reshape(NW, NCH, B)
    dst1 = jnp.where((dstf >= RPP) & (dstf < 2 * RPP), dstf - RPP,
                     GARB_LOCAL).reshape(NW, NCH, B)
    dst2 = jnp.where(dstf >= 2 * RPP, dstf - 2 * RPP,
                     GARB_LOCAL).reshape(NW, NCH, B)
    xp = jnp.pad(x, ((0, NPAD - N), (0, 0)))

    degp = _deg_kernel(dst)
    dpr = jnp.transpose(degp.reshape(NW, _G, _RB), (1, 0, 2))

    y1 = _tc_a(xp, W1, dpr)
    p = _agg1_kernel(y1, src, dst0, dst1, dst2)
    y2t = _tc_b(p, y1, dpr, b1.reshape(1, D), W2)
    q = _agg2_kernel(y2t.reshape(2 * NPAD), src, dst)
    outt = _tc_d(q, y2t, degp, b2.reshape(2, 1))
    return outt[:, :N].T


# speedup vs baseline: 7.9705x; 1.3508x over previous
import functools

import jax
import jax.numpy as jnp
from jax import lax
from jax.experimental import pallas as pl
from jax.experimental.pallas import tpu as pltpu
from jax.experimental.pallas import tpu_sc as plsc

N = 10000
D = 128
E = 320000

NC = 2
NS = 16
NW = NC * NS

B = 128
NCH = 80
E_PAD = NW * NCH * B
NPAD = 10240
GARBAGE = N + 100
NPASS = 3
RPP = 3456
ACC1_ROWS = RPP + 8
GARB_LOCAL = RPP
RPT1 = RPP // NS
OUT1_ROWS = NPASS * RPP
LLEN = (NCH + 4) * B
NBUF = 2

_mesh = plsc.VectorSubcoreMesh(core_axis_name="c", subcore_axis_name="s")


def _deg_body(dsts_hbm, degp_hbm, dst_v, acc, sem):
    cid = lax.axis_index("c")
    sid = lax.axis_index("s")
    wid = cid * NS + sid

    pltpu.sync_copy(dsts_hbm.at[wid], dst_v)

    def fill(i, _):
        acc[pl.ds(i * 16, 16)] = jnp.zeros((16,), jnp.float32)
        return 0

    lax.fori_loop(0, NPAD // 16, fill, 0)

    ones16 = jnp.ones((16,), jnp.float32)

    def chunk(j, _):
        for s in range(B // 16):
            idx = dst_v[j, pl.ds(s * 16, 16)]
            plsc.addupdate_scatter(acc, [idx], ones16)
        return 0

    lax.fori_loop(0, NCH, chunk, 0)
    pltpu.sync_copy(acc, degp_hbm.at[wid])


_deg_kernel = functools.partial(
    pl.kernel,
    out_type=jax.ShapeDtypeStruct((NW, NPAD), jnp.float32),
    mesh=_mesh,
    compiler_params=pltpu.CompilerParams(needs_layout_passes=False, internal_scratch_in_bytes=0),
    scratch_types=[
        pltpu.VMEM((NCH, B), jnp.int32),
        pltpu.VMEM((NPAD,), jnp.float32),
        pltpu.SemaphoreType.DMA,
    ],
)(_deg_body)


def _agg1_body(y_hbm, srcs_hbm, d0_hbm, d1_hbm, d2_hbm, out_hbm, src_v,
               dst_v, lsrc, ldst, lsrc2, ldst2, bufs, acc, sem):
    cid = lax.axis_index("c")
    sid = lax.axis_index("s")
    wid = cid * NS + sid

    pltpu.sync_copy(srcs_hbm.at[wid], src_v)

    lo = sid * RPT1
    for p, d_hbm in ((0, d0_hbm), (1, d1_hbm), (2, d2_hbm)):
        pltpu.sync_copy(d_hbm.at[wid], dst_v)

        def zfill(i, _):
            bufs[0, lax.div(i, 8), pl.ds(lax.rem(i, 8) * 16, 16)] = (
                jnp.zeros((16,), jnp.float32))
            return 0

        lax.fori_loop(0, B * 8, zfill, 0)
        for k, sz in ((0, B), (1, RPT1 - B)):
            pltpu.sync_copy(bufs.at[0].at[pl.ds(0, sz)],
                            acc.at[pl.ds(lo + k * B, sz)])

        def pfill(i, _):
            lsrc[pl.ds(i * 16, 16)] = jnp.zeros((16,), jnp.int32)
            ldst[pl.ds(i * 16, 16)] = jnp.full((16,), GARB_LOCAL, jnp.int32)
            return 0

        lax.fori_loop(0, LLEN // 16, pfill, 0)

        def comp(t, cnt):
            r = lax.div(t, B // 16)
            c = lax.rem(t, B // 16)
            vs = src_v[r, pl.ds(c * 16, 16)]
            vd = dst_v[r, pl.ds(c * 16, 16)]
            m = vd < GARB_LOCAL
            plsc.store_compressed(lsrc.at[pl.ds(cnt, 16)], vs, mask=m)
            plsc.store_compressed(ldst.at[pl.ds(cnt, 16)], vd, mask=m)
            return cnt + jnp.max(plsc.all_reduce_population_count(m))

        cnt = lax.fori_loop(0, NCH * (B // 16), comp, jnp.int32(0))
        nch4 = lax.div(cnt + (NBUF * B - 1), NBUF * B)

        def lcopy(j, _):
            for st in range(B // 16):
                lsrc2[j, pl.ds(st * 16, 16)] = lsrc[pl.ds(j * B + st * 16, 16)]
                ldst2[j, pl.ds(st * 16, 16)] = ldst[pl.ds(j * B + st * 16, 16)]
            return 0

        lax.fori_loop(0, nch4 * NBUF, lcopy, 0)
        plsc.subcore_barrier()

        def chunk4(i, _):
            j0 = i * NBUF
            hs = []
            for k in range(NBUF):
                hs.append(pltpu.async_copy(
                    y_hbm.at[lsrc2.at[j0 + k]], bufs.at[k], sem))
            for k in range(NBUF):
                hs[k].wait()
                pltpu.sync_copy(bufs.at[k], acc.at[ldst2.at[j0 + k]],
                                add=True)
            return 0

        lax.fori_loop(0, nch4, chunk4, 0)
        plsc.subcore_barrier()
        for k, sz in ((0, B), (1, RPT1 - B)):
            pltpu.sync_copy(acc.at[pl.ds(lo + k * B, sz)],
                            bufs.at[0].at[pl.ds(0, sz)])
            pltpu.sync_copy(
                bufs.at[0].at[pl.ds(0, sz)],
                out_hbm.at[cid, pl.ds(p * RPP + lo + k * B, sz)])
        plsc.subcore_barrier()


_agg1_kernel = functools.partial(
    pl.kernel,
    out_type=jax.ShapeDtypeStruct((NC, OUT1_ROWS, D), jnp.float32),
    mesh=_mesh,
    compiler_params=pltpu.CompilerParams(needs_layout_passes=False),
    scratch_types=[
        pltpu.VMEM((NCH, B), jnp.int32),
        pltpu.VMEM((NCH, B), jnp.int32),
        pltpu.VMEM((LLEN,), jnp.int32),
        pltpu.VMEM((LLEN,), jnp.int32),
        pltpu.VMEM((LLEN // B, B), jnp.int32),
        pltpu.VMEM((LLEN // B, B), jnp.int32),
        pltpu.VMEM((NBUF, B, D), jnp.float32),
        pltpu.VMEM_SHARED((ACC1_ROWS, D), jnp.float32),
        pltpu.SemaphoreType.DMA,
    ],
)(_agg1_body)


def _agg2_body(yt_hbm, srcs_hbm, dsts_hbm, out_hbm, src_v, dst_v, bufs, acc,
               sem):
    cid = lax.axis_index("c")
    sid = lax.axis_index("s")
    wid = cid * NS + sid

    pltpu.sync_copy(srcs_hbm.at[wid], src_v)
    pltpu.sync_copy(dsts_hbm.at[wid], dst_v)

    def zfill(i, _):
        for f in range(2):
            acc[f, pl.ds(i * 16, 16)] = jnp.zeros((16,), jnp.float32)
        return 0

    lax.fori_loop(0, NPAD // 16, zfill, 0)

    for f in range(2):
        yf = yt_hbm.at[pl.ds(f * NPAD, NPAD)]
        fidx = jnp.full((16,), f, jnp.int32)

        def chunk4(i, _):
            j0 = i * NBUF
            hs = []
            for k in range(NBUF):
                hs.append(pltpu.async_copy(
                    yf.at[src_v.at[j0 + k]], bufs.at[k], sem))
            for k in range(NBUF):
                hs[k].wait()
                for s in range(B // 16):
                    vals = bufs[k, pl.ds(s * 16, 16)]
                    idx = dst_v[j0 + k, pl.ds(s * 16, 16)]
                    plsc.addupdate_scatter(acc, [fidx, idx], vals)
            return 0

        lax.fori_loop(0, NCH // NBUF, chunk4, 0)
    pltpu.sync_copy(acc, out_hbm.at[wid])


_agg2_kernel = functools.partial(
    pl.kernel,
    out_type=jax.ShapeDtypeStruct((NW, 2, NPAD), jnp.float32),
    mesh=_mesh,
    compiler_params=pltpu.CompilerParams(needs_layout_passes=False, internal_scratch_in_bytes=0),
    scratch_types=[
        pltpu.VMEM((NCH, B), jnp.int32),
        pltpu.VMEM((NCH, B), jnp.int32),
        pltpu.VMEM((NBUF, B), jnp.float32),
        pltpu.VMEM((2, NPAD), jnp.float32),
        pltpu.SemaphoreType.DMA,
    ],
)(_agg2_body)


def _tc_a_body(x_ref, w_ref, dp_ref, y_ref):
    deg = jnp.sum(dp_ref[0], axis=0) + 1.0
    dis = lax.rsqrt(deg)
    xw = jnp.dot(x_ref[...], w_ref[...], preferred_element_type=jnp.float32)
    y_ref[...] = xw * dis[:, None]


def _tc_b_body(p_ref, y1_ref, dp_ref, b1_ref, w2_ref, y2t_ref):
    deg = jnp.sum(dp_ref[0], axis=0) + 1.0
    dis = lax.rsqrt(deg)
    agg = p_ref[0] + p_ref[1]
    h = dis[:, None] * (agg + y1_ref[...]) + b1_ref[...]
    h = jnp.maximum(h, 0.0)
    hw = jnp.dot(h, w2_ref[...], preferred_element_type=jnp.float32)
    y2t_ref[...] = (hw * dis[:, None]).T


def _tc_d_body(q_ref, y2t_ref, dp_ref, b2_ref, out_ref):
    deg = jnp.sum(dp_ref[...], axis=0) + 1.0
    dis = lax.rsqrt(deg)
    agg = jnp.sum(q_ref[...], axis=0)
    out_ref[...] = dis[None, :] * (agg + y2t_ref[...]) + b2_ref[...]


_RB = 1024
_G = NPAD // _RB


def _tc_a(xp, W1, dp):
    return pl.pallas_call(
        _tc_a_body,
        grid=(_G,),
        in_specs=[
            pl.BlockSpec((_RB, D), lambda i: (i, 0)),
            pl.BlockSpec((D, D), lambda i: (0, 0)),
            pl.BlockSpec((1, NW, _RB), lambda i: (i, 0, 0)),
        ],
        out_specs=pl.BlockSpec((_RB, D), lambda i: (i, 0)),
        out_shape=jax.ShapeDtypeStruct((NPAD, D), jnp.float32),
    )(xp, W1, dp)


def _tc_b(p, y1, dp, b1, W2):
    return pl.pallas_call(
        _tc_b_body,
        grid=(_G,),
        in_specs=[
            pl.BlockSpec((NC, _RB, D), lambda i: (0, i, 0)),
            pl.BlockSpec((_RB, D), lambda i: (i, 0)),
            pl.BlockSpec((1, NW, _RB), lambda i: (i, 0, 0)),
            pl.BlockSpec((1, D), lambda i: (0, 0)),
            pl.BlockSpec((D, 2), lambda i: (0, 0)),
        ],
        out_specs=pl.BlockSpec((2, _RB), lambda i: (0, i)),
        out_shape=jax.ShapeDtypeStruct((2, NPAD), jnp.float32),
    )(p, y1, dp, b1, W2)


def _tc_d(q, y2t, dp, b2):
    return pl.pallas_call(
        _tc_d_body,
        in_specs=[
            pl.BlockSpec((NW, 2, NPAD), lambda: (0, 0, 0)),
            pl.BlockSpec((2, NPAD), lambda: (0, 0)),
            pl.BlockSpec((NW, NPAD), lambda: (0, 0)),
            pl.BlockSpec((2, 1), lambda: (0, 0)),
        ],
        out_specs=pl.BlockSpec((2, NPAD), lambda: (0, 0)),
        out_shape=jax.ShapeDtypeStruct((2, NPAD), jnp.float32),
    )(q, y2t, dp, b2)


@jax.jit
def kernel(x, edge_index, W1, b1, W2, b2):
    ei = edge_index.astype(jnp.int32)
    srcf = jnp.concatenate([ei[0], jnp.zeros((E_PAD - E,), jnp.int32)])
    dstf = jnp.concatenate([ei[1], jnp.full((E_PAD - E,), GARBAGE, jnp.int32)])
    src = srcf.reshape(NW, NCH, B)
    dst = dstf.reshape(NW, NCH, B)
    dst0 = jnp.where(dstf < RPP, dstf, GARB_LOCAL).reshape(NW, NCH, B)
    dst1 = jnp.where((dstf >= RPP) & (dstf < 2 * RPP), dstf - RPP,
                     GARB_LOCAL).reshape(NW, NCH, B)
    dst2 = jnp.where(dstf >= 2 * RPP, dstf - 2 * RPP,
                     GARB_LOCAL).reshape(NW, NCH, B)
    xp = jnp.pad(x, ((0, NPAD - N), (0, 0)))

    degp = _deg_kernel(dst)
    dpr = jnp.transpose(degp.reshape(NW, _G, _RB), (1, 0, 2))

    y1 = _tc_a(xp, W1, dpr)
    p = _agg1_kernel(y1, src, dst0, dst1, dst2)
    y2t = _tc_b(p, y1, dpr, b1.reshape(1, D), W2)
    q = _agg2_kernel(y2t.reshape(2 * NPAD), src, dst)
    outt = _tc_d(q, y2t, degp, b2.reshape(2, 1))
    return outt[:, :N].T


# speedup vs baseline: 7.9991x; 1.0036x over previous
import functools

import jax
import jax.numpy as jnp
from jax import lax
from jax.experimental import pallas as pl
from jax.experimental.pallas import tpu as pltpu
from jax.experimental.pallas import tpu_sc as plsc

N = 10000
D = 128
E = 320000

NC = 2
NS = 16
NW = NC * NS

B = 128
NCH = 80
E_PAD = NW * NCH * B
NPAD = 10240
GARBAGE = N + 100
NPASS = 3
RPP = 3456
ACC1_ROWS = RPP + 8
GARB_LOCAL = RPP
RPT1 = RPP // NS
OUT1_ROWS = NPASS * RPP
C0 = 56
NCHX = NCH - C0
LLEN = (NCH + NCHX + 4) * B
NBUF = 2

_mesh = plsc.VectorSubcoreMesh(core_axis_name="c", subcore_axis_name="s")


def _deg_body(dsts_hbm, degp_hbm, dst_v, acc, sem):
    cid = lax.axis_index("c")
    sid = lax.axis_index("s")
    wid = cid * NS + sid

    pltpu.sync_copy(dsts_hbm.at[wid], dst_v)

    def fill(i, _):
        acc[pl.ds(i * 16, 16)] = jnp.zeros((16,), jnp.float32)
        return 0

    lax.fori_loop(0, NPAD // 16, fill, 0)

    ones16 = jnp.ones((16,), jnp.float32)

    def chunk(j, _):
        for s in range(B // 16):
            idx = dst_v[j, pl.ds(s * 16, 16)]
            plsc.addupdate_scatter(acc, [idx], ones16)
        return 0

    lax.fori_loop(0, NCH, chunk, 0)
    pltpu.sync_copy(acc, degp_hbm.at[wid])


_deg_kernel = functools.partial(
    pl.kernel,
    out_type=jax.ShapeDtypeStruct((NW, NPAD), jnp.float32),
    mesh=_mesh,
    compiler_params=pltpu.CompilerParams(needs_layout_passes=False, internal_scratch_in_bytes=0),
    scratch_types=[
        pltpu.VMEM((NCH, B), jnp.int32),
        pltpu.VMEM((NPAD,), jnp.float32),
        pltpu.SemaphoreType.DMA,
    ],
)(_deg_body)


def _agg1_body(y_hbm, srcs_hbm, d0_hbm, d1_hbm, d2_hbm, out_hbm, src_v,
               dst_v, srcx_v, dstx_v, lsrc, ldst, ldst2, bufs, acc, sem):
    cid = lax.axis_index("c")
    sid = lax.axis_index("s")
    wid = cid * NS + sid

    pltpu.sync_copy(srcs_hbm.at[wid], src_v)

    @pl.when(cid == 1)
    def _():
        pltpu.sync_copy(srcs_hbm.at[wid - NS].at[pl.ds(C0, NCHX)], srcx_v)

    n_main = jnp.where(cid == 0, C0 * (B // 16), NCH * (B // 16))
    n_x = jnp.where(cid == 0, 0, NCHX * (B // 16))

    lo = sid * RPT1
    for p, d_hbm in ((0, d0_hbm), (1, d1_hbm), (2, d2_hbm)):
        pltpu.sync_copy(d_hbm.at[wid], dst_v)

        @pl.when(cid == 1)
        def _():
            pltpu.sync_copy(d_hbm.at[wid - NS].at[pl.ds(C0, NCHX)], dstx_v)

        def zfill(i, _):
            bufs[0, lax.div(i, 8), pl.ds(lax.rem(i, 8) * 16, 16)] = (
                jnp.zeros((16,), jnp.float32))
            return 0

        lax.fori_loop(0, B * 8, zfill, 0)
        for k, sz in ((0, B), (1, RPT1 - B)):
            pltpu.sync_copy(bufs.at[0].at[pl.ds(0, sz)],
                            acc.at[pl.ds(lo + k * B, sz)])

        def pfill(i, _):
            lsrc[pl.ds(i * 16, 16)] = jnp.zeros((16,), jnp.int32)
            ldst[pl.ds(i * 16, 16)] = jnp.full((16,), GARB_LOCAL, jnp.int32)
            return 0

        lax.fori_loop(0, LLEN // 16, pfill, 0)

        def comp(t, cnt):
            r = lax.div(t, B // 16)
            c = lax.rem(t, B // 16)
            vs = src_v[r, pl.ds(c * 16, 16)]
            vd = dst_v[r, pl.ds(c * 16, 16)]
            m = vd < GARB_LOCAL
            plsc.store_compressed(lsrc.at[pl.ds(cnt, 16)], vs, mask=m)
            plsc.store_compressed(ldst.at[pl.ds(cnt, 16)], vd, mask=m)
            return cnt + jnp.max(plsc.all_reduce_population_count(m))

        cnt = lax.fori_loop(0, n_main, comp, jnp.int32(0))

        def compx(t, cnt):
            r = lax.div(t, B // 16)
            c = lax.rem(t, B // 16)
            vs = srcx_v[r, pl.ds(c * 16, 16)]
            vd = dstx_v[r, pl.ds(c * 16, 16)]
            m = vd < GARB_LOCAL
            plsc.store_compressed(lsrc.at[pl.ds(cnt, 16)], vs, mask=m)
            plsc.store_compressed(ldst.at[pl.ds(cnt, 16)], vd, mask=m)
            return cnt + jnp.max(plsc.all_reduce_population_count(m))

        cnt = lax.fori_loop(0, n_x, compx, cnt)
        nch4 = lax.div(cnt + (NBUF * B - 1), NBUF * B)

        def lcopy(j, _):
            for st in range(B // 16):
                ldst2[j, pl.ds(st * 16, 16)] = ldst[pl.ds(j * B + st * 16, 16)]
            return 0

        lax.fori_loop(0, nch4 * NBUF, lcopy, 0)
        plsc.subcore_barrier()

        def chunk4(i, _):
            j0 = i * NBUF
            hs = []
            for k in range(NBUF):
                hs.append(pltpu.async_copy(
                    y_hbm.at[lsrc.at[pl.ds((j0 + k) * B, B)]], bufs.at[k],
                    sem))
            for k in range(NBUF):
                hs[k].wait()
                pltpu.sync_copy(bufs.at[k], acc.at[ldst2.at[j0 + k]],
                                add=True)
            return 0

        lax.fori_loop(0, nch4, chunk4, 0)
        plsc.subcore_barrier()
        for k, sz in ((0, B), (1, RPT1 - B)):
            pltpu.sync_copy(acc.at[pl.ds(lo + k * B, sz)],
                            bufs.at[0].at[pl.ds(0, sz)])
            pltpu.sync_copy(
                bufs.at[0].at[pl.ds(0, sz)],
                out_hbm.at[cid, pl.ds(p * RPP + lo + k * B, sz)])
        plsc.subcore_barrier()


_agg1_kernel = functools.partial(
    pl.kernel,
    out_type=jax.ShapeDtypeStruct((NC, OUT1_ROWS, D), jnp.float32),
    mesh=_mesh,
    compiler_params=pltpu.CompilerParams(needs_layout_passes=False),
    scratch_types=[
        pltpu.VMEM((NCH, B), jnp.int32),
        pltpu.VMEM((NCH, B), jnp.int32),
        pltpu.VMEM((NCHX, B), jnp.int32),
        pltpu.VMEM((NCHX, B), jnp.int32),
        pltpu.VMEM((LLEN,), jnp.int32),
        pltpu.VMEM((LLEN,), jnp.int32),
        pltpu.VMEM((LLEN // B, B), jnp.int32),
        pltpu.VMEM((NBUF, B, D), jnp.float32),
        pltpu.VMEM_SHARED((ACC1_ROWS, D), jnp.float32),
        pltpu.SemaphoreType.DMA,
    ],
)(_agg1_body)


def _agg2_body(yt_hbm, srcs_hbm, dsts_hbm, out_hbm, src_v, dst_v, bufs, acc,
               sem):
    cid = lax.axis_index("c")
    sid = lax.axis_index("s")
    wid = cid * NS + sid

    pltpu.sync_copy(srcs_hbm.at[wid], src_v)
    pltpu.sync_copy(dsts_hbm.at[wid], dst_v)

    def zfill(i, _):
        for f in range(2):
            acc[f, pl.ds(i * 16, 16)] = jnp.zeros((16,), jnp.float32)
        return 0

    lax.fori_loop(0, NPAD // 16, zfill, 0)

    for f in range(2):
        yf = yt_hbm.at[pl.ds(f * NPAD, NPAD)]
        fidx = jnp.full((16,), f, jnp.int32)

        def chunk4(i, _):
            j0 = i * NBUF
            hs = []
            for k in range(NBUF):
                hs.append(pltpu.async_copy(
                    yf.at[src_v.at[j0 + k]], bufs.at[k], sem))
            for k in range(NBUF):
                hs[k].wait()
                for s in range(B // 16):
                    vals = bufs[k, pl.ds(s * 16, 16)]
                    idx = dst_v[j0 + k, pl.ds(s * 16, 16)]
                    plsc.addupdate_scatter(acc, [fidx, idx], vals)
            return 0

        lax.fori_loop(0, NCH // NBUF, chunk4, 0)
    pltpu.sync_copy(acc, out_hbm.at[wid])


_agg2_kernel = functools.partial(
    pl.kernel,
    out_type=jax.ShapeDtypeStruct((NW, 2, NPAD), jnp.float32),
    mesh=_mesh,
    compiler_params=pltpu.CompilerParams(needs_layout_passes=False, internal_scratch_in_bytes=0),
    scratch_types=[
        pltpu.VMEM((NCH, B), jnp.int32),
        pltpu.VMEM((NCH, B), jnp.int32),
        pltpu.VMEM((NBUF, B), jnp.float32),
        pltpu.VMEM((2, NPAD), jnp.float32),
        pltpu.SemaphoreType.DMA,
    ],
)(_agg2_body)


def _tc_a_body(x_ref, w_ref, dp_ref, y_ref):
    deg = jnp.sum(dp_ref[0], axis=0) + 1.0
    dis = lax.rsqrt(deg)
    xw = jnp.dot(x_ref[...], w_ref[...], preferred_element_type=jnp.float32)
    y_ref[...] = xw * dis[:, None]


def _tc_b_body(p_ref, y1_ref, dp_ref, b1_ref, w2_ref, y2t_ref):
    deg = jnp.sum(dp_ref[0], axis=0) + 1.0
    dis = lax.rsqrt(deg)
    agg = p_ref[0] + p_ref[1]
    h = dis[:, None] * (agg + y1_ref[...]) + b1_ref[...]
    h = jnp.maximum(h, 0.0)
    hw = jnp.dot(h, w2_ref[...], preferred_element_type=jnp.float32)
    y2t_ref[...] = (hw * dis[:, None]).T


def _tc_d_body(q_ref, y2t_ref, dp_ref, b2_ref, out_ref):
    deg = jnp.sum(dp_ref[...], axis=0) + 1.0
    dis = lax.rsqrt(deg)
    agg = jnp.sum(q_ref[...], axis=0)
    out_ref[...] = dis[None, :] * (agg + y2t_ref[...]) + b2_ref[...]


_RB = 1024
_G = NPAD // _RB


def _tc_a(xp, W1, dp):
    return pl.pallas_call(
        _tc_a_body,
        grid=(_G,),
        in_specs=[
            pl.BlockSpec((_RB, D), lambda i: (i, 0)),
            pl.BlockSpec((D, D), lambda i: (0, 0)),
            pl.BlockSpec((1, NW, _RB), lambda i: (i, 0, 0)),
        ],
        out_specs=pl.BlockSpec((_RB, D), lambda i: (i, 0)),
        out_shape=jax.ShapeDtypeStruct((NPAD, D), jnp.float32),
    )(xp, W1, dp)


def _tc_b(p, y1, dp, b1, W2):
    return pl.pallas_call(
        _tc_b_body,
        grid=(_G,),
        in_specs=[
            pl.BlockSpec((NC, _RB, D), lambda i: (0, i, 0)),
            pl.BlockSpec((_RB, D), lambda i: (i, 0)),
            pl.BlockSpec((1, NW, _RB), lambda i: (i, 0, 0)),
            pl.BlockSpec((1, D), lambda i: (0, 0)),
            pl.BlockSpec((D, 2), lambda i: (0, 0)),
        ],
        out_specs=pl.BlockSpec((2, _RB), lambda i: (0, i)),
        out_shape=jax.ShapeDtypeStruct((2, NPAD), jnp.float32),
    )(p, y1, dp, b1, W2)


def _tc_d(q, y2t, dp, b2):
    return pl.pallas_call(
        _tc_d_body,
        in_specs=[
            pl.BlockSpec((NW, 2, NPAD), lambda: (0, 0, 0)),
            pl.BlockSpec((2, NPAD), lambda: (0, 0)),
            pl.BlockSpec((NW, NPAD), lambda: (0, 0)),
            pl.BlockSpec((2, 1), lambda: (0, 0)),
        ],
        out_specs=pl.BlockSpec((2, NPAD), lambda: (0, 0)),
        out_shape=jax.ShapeDtypeStruct((2, NPAD), jnp.float32),
    )(q, y2t, dp, b2)


@jax.jit
def kernel(x, edge_index, W1, b1, W2, b2):
    ei = edge_index.astype(jnp.int32)
    srcf = jnp.concatenate([ei[0], jnp.zeros((E_PAD - E,), jnp.int32)])
    dstf = jnp.concatenate([ei[1], jnp.full((E_PAD - E,), GARBAGE, jnp.int32)])
    src = srcf.reshape(NW, NCH, B)
    dst = dstf.reshape(NW, NCH, B)
    dst0 = jnp.where(dstf < RPP, dstf, GARB_LOCAL).reshape(NW, NCH, B)
    dst1 = jnp.where((dstf >= RPP) & (dstf < 2 * RPP), dstf - RPP,
                     GARB_LOCAL).reshape(NW, NCH, B)
    dst2 = jnp.where(dstf >= 2 * RPP, dstf - 2 * RPP,
                     GARB_LOCAL).reshape(NW, NCH, B)
    xp = jnp.pad(x, ((0, NPAD - N), (0, 0)))

    degp = _deg_kernel(dst)
    dpr = jnp.transpose(degp.reshape(NW, _G, _RB), (1, 0, 2))

    y1 = _tc_a(xp, W1, dpr)
    p = _agg1_kernel(y1, src, dst0, dst1, dst2)
    y2t = _tc_b(p, y1, dpr, b1.reshape(1, D), W2)
    q = _agg2_kernel(y2t.reshape(2 * NPAD), src, dst)
    outt = _tc_d(q, y2t, degp, b2.reshape(2, 1))
    return outt[:, :N].T


# speedup vs baseline: 9.3368x; 1.1672x over previous
import functools

import jax
import jax.numpy as jnp
from jax import lax
from jax.experimental import pallas as pl
from jax.experimental.pallas import tpu as pltpu
from jax.experimental.pallas import tpu_sc as plsc

N = 10000
D = 128
E = 320000

NC = 2
NS = 16
NW = NC * NS

B = 128
NCH = 80
E_PAD = NW * NCH * B
NPAD = 10240
GARBAGE = N + 100
NPASS = 3
RPP = 3456
ACC1_ROWS = RPP + 8
GARB_LOCAL = RPP
RPT1 = RPP // NS
OUT1_ROWS = NPASS * RPP
C0 = 56
NCHX = NCH - C0
LLEN = (NCH + NCHX + 4) * B
NBUF = 2

_mesh = plsc.VectorSubcoreMesh(core_axis_name="c", subcore_axis_name="s")


def _deg_body(dsts_hbm, degp_hbm, dst_v, acc, sem):
    cid = lax.axis_index("c")
    sid = lax.axis_index("s")
    wid = cid * NS + sid

    pltpu.sync_copy(dsts_hbm.at[wid], dst_v)

    def fill(i, _):
        acc[pl.ds(i * 16, 16)] = jnp.zeros((16,), jnp.float32)
        return 0

    lax.fori_loop(0, NPAD // 16, fill, 0)

    ones16 = jnp.ones((16,), jnp.float32)

    def chunk(j, _):
        for s in range(B // 16):
            idx = dst_v[j, pl.ds(s * 16, 16)]
            plsc.addupdate_scatter(acc, [idx], ones16)
        return 0

    lax.fori_loop(0, NCH, chunk, 0)
    pltpu.sync_copy(acc, degp_hbm.at[wid])


_deg_kernel = functools.partial(
    pl.kernel,
    out_type=jax.ShapeDtypeStruct((NW, NPAD), jnp.float32),
    mesh=_mesh,
    compiler_params=pltpu.CompilerParams(needs_layout_passes=False, internal_scratch_in_bytes=0),
    scratch_types=[
        pltpu.VMEM((NCH, B), jnp.int32),
        pltpu.VMEM((NPAD,), jnp.float32),
        pltpu.SemaphoreType.DMA,
    ],
)(_deg_body)


def _agg1_body(y_hbm, srcs_hbm, d0_hbm, d1_hbm, d2_hbm, out_hbm, src_v,
               dst_v, srcx_v, dstx_v, lsrc, ldst, ldst2, bufs, acc, sem):
    cid = lax.axis_index("c")
    sid = lax.axis_index("s")
    wid = cid * NS + sid

    pltpu.sync_copy(srcs_hbm.at[wid], src_v)

    @pl.when(cid == 0)
    def _():
        pltpu.sync_copy(srcs_hbm.at[wid + NS].at[pl.ds(C0, NCHX)], srcx_v)

    n_main = jnp.where(cid == 1, C0 * (B // 16), NCH * (B // 16))
    n_x = jnp.where(cid == 1, 0, NCHX * (B // 16))

    lo = sid * RPT1
    for p, d_hbm in ((0, d0_hbm), (1, d1_hbm), (2, d2_hbm)):
        pltpu.sync_copy(d_hbm.at[wid], dst_v)

        @pl.when(cid == 0)
        def _():
            pltpu.sync_copy(d_hbm.at[wid + NS].at[pl.ds(C0, NCHX)], dstx_v)

        def zfill(i, _):
            bufs[0, lax.div(i, 8), pl.ds(lax.rem(i, 8) * 16, 16)] = (
                jnp.zeros((16,), jnp.float32))
            return 0

        lax.fori_loop(0, B * 8, zfill, 0)
        for k, sz in ((0, B), (1, RPT1 - B)):
            pltpu.sync_copy(bufs.at[0].at[pl.ds(0, sz)],
                            acc.at[pl.ds(lo + k * B, sz)])

        def pfill(i, _):
            lsrc[pl.ds(i * 16, 16)] = jnp.zeros((16,), jnp.int32)
            ldst[pl.ds(i * 16, 16)] = jnp.full((16,), GARB_LOCAL, jnp.int32)
            return 0

        lax.fori_loop(0, LLEN // 16, pfill, 0)

        def comp(t, cnt):
            r = lax.div(t, B // 16)
            c = lax.rem(t, B // 16)
            vs = src_v[r, pl.ds(c * 16, 16)]
            vd = dst_v[r, pl.ds(c * 16, 16)]
            m = vd < GARB_LOCAL
            plsc.store_compressed(lsrc.at[pl.ds(cnt, 16)], vs, mask=m)
            plsc.store_compressed(ldst.at[pl.ds(cnt, 16)], vd, mask=m)
            return cnt + jnp.max(plsc.all_reduce_population_count(m))

        cnt = lax.fori_loop(0, n_main, comp, jnp.int32(0))

        def compx(t, cnt):
            r = lax.div(t, B // 16)
            c = lax.rem(t, B // 16)
            vs = srcx_v[r, pl.ds(c * 16, 16)]
            vd = dstx_v[r, pl.ds(c * 16, 16)]
            m = vd < GARB_LOCAL
            plsc.store_compressed(lsrc.at[pl.ds(cnt, 16)], vs, mask=m)
            plsc.store_compressed(ldst.at[pl.ds(cnt, 16)], vd, mask=m)
            return cnt + jnp.max(plsc.all_reduce_population_count(m))

        cnt = lax.fori_loop(0, n_x, compx, cnt)
        nch4 = lax.div(cnt + (NBUF * B - 1), NBUF * B)

        def lcopy(j, _):
            for st in range(B // 16):
                ldst2[j, pl.ds(st * 16, 16)] = ldst[pl.ds(j * B + st * 16, 16)]
            return 0

        lax.fori_loop(0, nch4 * NBUF, lcopy, 0)
        plsc.subcore_barrier()

        def chunk4(i, _):
            j0 = i * NBUF
            hs = []
            for k in range(NBUF):
                hs.append(pltpu.async_copy(
                    y_hbm.at[lsrc.at[pl.ds((j0 + k) * B, B)]], bufs.at[k],
                    sem))
            for k in range(NBUF):
                hs[k].wait()
                pltpu.sync_copy(bufs.at[k], acc.at[ldst2.at[j0 + k]],
                                add=True)
            return 0

        lax.fori_loop(0, nch4, chunk4, 0)
        plsc.subcore_barrier()
        for k, sz in ((0, B), (1, RPT1 - B)):
            pltpu.sync_copy(acc.at[pl.ds(lo + k * B, sz)],
                            bufs.at[0].at[pl.ds(0, sz)])
            pltpu.sync_copy(
                bufs.at[0].at[pl.ds(0, sz)],
                out_hbm.at[cid, pl.ds(p * RPP + lo + k * B, sz)])
        plsc.subcore_barrier()


_agg1_kernel = functools.partial(
    pl.kernel,
    out_type=jax.ShapeDtypeStruct((NC, OUT1_ROWS, D), jnp.float32),
    mesh=_mesh,
    compiler_params=pltpu.CompilerParams(needs_layout_passes=False),
    scratch_types=[
        pltpu.VMEM((NCH, B), jnp.int32),
        pltpu.VMEM((NCH, B), jnp.int32),
        pltpu.VMEM((NCHX, B), jnp.int32),
        pltpu.VMEM((NCHX, B), jnp.int32),
        pltpu.VMEM((LLEN,), jnp.int32),
        pltpu.VMEM((LLEN,), jnp.int32),
        pltpu.VMEM((LLEN // B, B), jnp.int32),
        pltpu.VMEM((NBUF, B, D), jnp.float32),
        pltpu.VMEM_SHARED((ACC1_ROWS, D), jnp.float32),
        pltpu.SemaphoreType.DMA,
    ],
)(_agg1_body)


def _agg2_body(yt_hbm, srcs_hbm, dsts_hbm, out_hbm, src_v, dst_v, bufs, acc,
               sem):
    cid = lax.axis_index("c")
    sid = lax.axis_index("s")
    wid = cid * NS + sid

    pltpu.sync_copy(srcs_hbm.at[wid], src_v)
    pltpu.sync_copy(dsts_hbm.at[wid], dst_v)

    def zfill(i, _):
        for f in range(2):
            acc[f, pl.ds(i * 16, 16)] = jnp.zeros((16,), jnp.float32)
        return 0

    lax.fori_loop(0, NPAD // 16, zfill, 0)

    for f in range(2):
        yf = yt_hbm.at[pl.ds(f * NPAD, NPAD)]
        fidx = jnp.full((16,), f, jnp.int32)

        def chunk4(i, _):
            j0 = i * NBUF
            hs = []
            for k in range(NBUF):
                hs.append(pltpu.async_copy(
                    yf.at[src_v.at[j0 + k]], bufs.at[k], sem))
            for k in range(NBUF):
                hs[k].wait()
                for s in range(B // 16):
                    vals = bufs[k, pl.ds(s * 16, 16)]
                    idx = dst_v[j0 + k, pl.ds(s * 16, 16)]
                    plsc.addupdate_scatter(acc, [fidx, idx], vals)
            return 0

        lax.fori_loop(0, NCH // NBUF, chunk4, 0)
    pltpu.sync_copy(acc, out_hbm.at[wid])


_agg2_kernel = functools.partial(
    pl.kernel,
    out_type=jax.ShapeDtypeStruct((NW, 2, NPAD), jnp.float32),
    mesh=_mesh,
    compiler_params=pltpu.CompilerParams(needs_layout_passes=False, internal_scratch_in_bytes=0),
    scratch_types=[
        pltpu.VMEM((NCH, B), jnp.int32),
        pltpu.VMEM((NCH, B), jnp.int32),
        pltpu.VMEM((NBUF, B), jnp.float32),
        pltpu.VMEM((2, NPAD), jnp.float32),
        pltpu.SemaphoreType.DMA,
    ],
)(_agg2_body)


def _tc_a_body(x_ref, w_ref, dp_ref, y_ref):
    deg = jnp.sum(dp_ref[0], axis=0) + 1.0
    dis = lax.rsqrt(deg)
    xw = jnp.dot(x_ref[...], w_ref[...], preferred_element_type=jnp.float32)
    y_ref[...] = xw * dis[:, None]


def _tc_b_body(p_ref, y1_ref, dp_ref, b1_ref, w2_ref, y2t_ref):
    deg = jnp.sum(dp_ref[0], axis=0) + 1.0
    dis = lax.rsqrt(deg)
    agg = p_ref[0] + p_ref[1]
    h = dis[:, None] * (agg + y1_ref[...]) + b1_ref[...]
    h = jnp.maximum(h, 0.0)
    hw = jnp.dot(h, w2_ref[...], preferred_element_type=jnp.float32)
    y2t_ref[...] = (hw * dis[:, None]).T


def _tc_d_body(q_ref, y2t_ref, dp_ref, b2_ref, out_ref):
    deg = jnp.sum(dp_ref[...], axis=0) + 1.0
    dis = lax.rsqrt(deg)
    agg = jnp.sum(q_ref[...], axis=0)
    out_ref[...] = dis[None, :] * (agg + y2t_ref[...]) + b2_ref[...]


_RB = 1024
_G = NPAD // _RB


def _tc_a(xp, W1, dp):
    return pl.pallas_call(
        _tc_a_body,
        grid=(_G,),
        in_specs=[
            pl.BlockSpec((_RB, D), lambda i: (i, 0)),
            pl.BlockSpec((D, D), lambda i: (0, 0)),
            pl.BlockSpec((1, NW, _RB), lambda i: (i, 0, 0)),
        ],
        out_specs=pl.BlockSpec((_RB, D), lambda i: (i, 0)),
        out_shape=jax.ShapeDtypeStruct((NPAD, D), jnp.float32),
    )(xp, W1, dp)


def _tc_b(p, y1, dp, b1, W2):
    return pl.pallas_call(
        _tc_b_body,
        grid=(_G,),
        in_specs=[
            pl.BlockSpec((NC, _RB, D), lambda i: (0, i, 0)),
            pl.BlockSpec((_RB, D), lambda i: (i, 0)),
            pl.BlockSpec((1, NW, _RB), lambda i: (i, 0, 0)),
            pl.BlockSpec((1, D), lambda i: (0, 0)),
            pl.BlockSpec((D, 2), lambda i: (0, 0)),
        ],
        out_specs=pl.BlockSpec((2, _RB), lambda i: (0, i)),
        out_shape=jax.ShapeDtypeStruct((2, NPAD), jnp.float32),
    )(p, y1, dp, b1, W2)


def _tc_d(q, y2t, dp, b2):
    return pl.pallas_call(
        _tc_d_body,
        in_specs=[
            pl.BlockSpec((NW, 2, NPAD), lambda: (0, 0, 0)),
            pl.BlockSpec((2, NPAD), lambda: (0, 0)),
            pl.BlockSpec((NW, NPAD), lambda: (0, 0)),
            pl.BlockSpec((2, 1), lambda: (0, 0)),
        ],
        out_specs=pl.BlockSpec((2, NPAD), lambda: (0, 0)),
        out_shape=jax.ShapeDtypeStruct((2, NPAD), jnp.float32),
    )(q, y2t, dp, b2)


@jax.jit
def kernel(x, edge_index, W1, b1, W2, b2):
    ei = edge_index.astype(jnp.int32)
    srcf = jnp.concatenate([ei[0], jnp.zeros((E_PAD - E,), jnp.int32)])
    dstf = jnp.concatenate([ei[1], jnp.full((E_PAD - E,), GARBAGE, jnp.int32)])
    src = srcf.reshape(NW, NCH, B)
    dst = dstf.reshape(NW, NCH, B)
    dst0 = jnp.where(dstf < RPP, dstf, GARB_LOCAL).reshape(NW, NCH, B)
    dst1 = jnp.where((dstf >= RPP) & (dstf < 2 * RPP), dstf - RPP,
                     GARB_LOCAL).reshape(NW, NCH, B)
    dst2 = jnp.where(dstf >= 2 * RPP, dstf - 2 * RPP,
                     GARB_LOCAL).reshape(NW, NCH, B)
    xp = jnp.pad(x, ((0, NPAD - N), (0, 0)))

    degp = _deg_kernel(dst)
    dpr = jnp.transpose(degp.reshape(NW, _G, _RB), (1, 0, 2))

    y1 = _tc_a(xp, W1, dpr)
    p = _agg1_kernel(y1, src, dst0, dst1, dst2)
    y2t = _tc_b(p, y1, dpr, b1.reshape(1, D), W2)
    q = _agg2_kernel(y2t.reshape(2 * NPAD), src, dst)
    outt = _tc_d(q, y2t, degp, b2.reshape(2, 1))
    return outt[:, :N].T


# speedup vs baseline: 9.4798x; 1.0153x over previous
import functools

import jax
import jax.numpy as jnp
from jax import lax
from jax.experimental import pallas as pl
from jax.experimental.pallas import tpu as pltpu
from jax.experimental.pallas import tpu_sc as plsc

N = 10000
D = 128
E = 320000

NC = 2
NS = 16
NW = NC * NS

B = 128
NCH = 80
E_PAD = NW * NCH * B
NPAD = 10240
GARBAGE = N + 100
NPASS = 3
RPP = 3456
ACC1_ROWS = RPP + 8
GARB_LOCAL = RPP
RPT1 = RPP // NS
OUT1_ROWS = NPASS * RPP
C0 = 56
NCHX = NCH - C0
LLEN = (NCH + NCHX + 4) * B
NBUF = 2

_mesh = plsc.VectorSubcoreMesh(core_axis_name="c", subcore_axis_name="s")


def _deg_body(dsts_hbm, degp_hbm, dst_v, acc, sem):
    cid = lax.axis_index("c")
    sid = lax.axis_index("s")
    wid = cid * NS + sid

    pltpu.sync_copy(dsts_hbm.at[wid], dst_v)

    def fill(i, _):
        acc[pl.ds(i * 16, 16)] = jnp.zeros((16,), jnp.float32)
        return 0

    lax.fori_loop(0, NPAD // 16, fill, 0)

    ones16 = jnp.ones((16,), jnp.float32)

    def chunk(j, _):
        for s in range(B // 16):
            idx = dst_v[j, pl.ds(s * 16, 16)]
            plsc.addupdate_scatter(acc, [idx], ones16)
        return 0

    lax.fori_loop(0, NCH, chunk, 0)
    pltpu.sync_copy(acc, degp_hbm.at[wid])


_deg_kernel = functools.partial(
    pl.kernel,
    out_type=jax.ShapeDtypeStruct((NW, NPAD), jnp.float32),
    mesh=_mesh,
    compiler_params=pltpu.CompilerParams(needs_layout_passes=False, internal_scratch_in_bytes=0),
    scratch_types=[
        pltpu.VMEM((NCH, B), jnp.int32),
        pltpu.VMEM((NPAD,), jnp.float32),
        pltpu.SemaphoreType.DMA,
    ],
)(_deg_body)


def _agg1_body(y_hbm, p0_hbm, p1_hbm, p2_hbm, out_hbm, pk_v, pkx_v, lsrc,
               ldst, ldst2, bufs, acc, sem, sems):
    cid = lax.axis_index("c")
    sid = lax.axis_index("s")
    wid = cid * NS + sid

    n_main = jnp.where(cid == 1, C0 * (B // 16), NCH * (B // 16))
    n_x = jnp.where(cid == 1, 0, NCHX * (B // 16))

    lo = sid * RPT1
    for p, p_hbm in ((0, p0_hbm), (1, p1_hbm), (2, p2_hbm)):
        pltpu.sync_copy(p_hbm.at[wid], pk_v)

        @pl.when(cid == 0)
        def _():
            pltpu.sync_copy(p_hbm.at[wid + NS].at[pl.ds(C0, NCHX)], pkx_v)

        def zfill(i, _):
            bufs[0, lax.div(i, 8), pl.ds(lax.rem(i, 8) * 16, 16)] = (
                jnp.zeros((16,), jnp.float32))
            return 0

        lax.fori_loop(0, B * 8, zfill, 0)
        for k, sz in ((0, B), (1, RPT1 - B)):
            pltpu.sync_copy(bufs.at[0].at[pl.ds(0, sz)],
                            acc.at[pl.ds(lo + k * B, sz)])

        def pfill(i, _):
            lsrc[pl.ds(i * 16, 16)] = jnp.zeros((16,), jnp.int32)
            ldst[pl.ds(i * 16, 16)] = jnp.full((16,), GARB_LOCAL, jnp.int32)
            return 0

        lax.fori_loop(0, LLEN // 16, pfill, 0)

        def comp(t, cnt):
            r = lax.div(t, B // 16)
            c = lax.rem(t, B // 16)
            v = pk_v[r, pl.ds(c * 16, 16)]
            vs = lax.shift_right_logical(v, 12)
            vd = lax.bitwise_and(v, 4095)
            m = vd < GARB_LOCAL
            plsc.store_compressed(lsrc.at[pl.ds(cnt, 16)], vs, mask=m)
            plsc.store_compressed(ldst.at[pl.ds(cnt, 16)], vd, mask=m)
            return cnt + jnp.max(plsc.all_reduce_population_count(m))

        cnt = lax.fori_loop(0, n_main, comp, jnp.int32(0))

        def compx(t, cnt):
            r = lax.div(t, B // 16)
            c = lax.rem(t, B // 16)
            v = pkx_v[r, pl.ds(c * 16, 16)]
            vs = lax.shift_right_logical(v, 12)
            vd = lax.bitwise_and(v, 4095)
            m = vd < GARB_LOCAL
            plsc.store_compressed(lsrc.at[pl.ds(cnt, 16)], vs, mask=m)
            plsc.store_compressed(ldst.at[pl.ds(cnt, 16)], vd, mask=m)
            return cnt + jnp.max(plsc.all_reduce_population_count(m))

        cnt = lax.fori_loop(0, n_x, compx, cnt)
        nch4 = lax.div(cnt + (NBUF * B - 1), NBUF * B)

        def lcopy(j, _):
            for st in range(B // 16):
                ldst2[j, pl.ds(st * 16, 16)] = ldst[pl.ds(j * B + st * 16, 16)]
            return 0

        lax.fori_loop(0, nch4 * NBUF, lcopy, 0)
        plsc.subcore_barrier()

        def chunk4(i, _):
            j0 = i * NBUF
            hs = []
            for k in range(NBUF):
                hs.append(pltpu.async_copy(
                    y_hbm.at[lsrc.at[pl.ds((j0 + k) * B, B)]], bufs.at[k],
                    sem))
            ss = []
            for k in range(NBUF):
                hs[k].wait()
                ss.append(pltpu.async_copy(
                    bufs.at[k], acc.at[ldst2.at[j0 + k]], sems, add=True))
            for k in range(NBUF):
                ss[k].wait()
            return 0

        lax.fori_loop(0, nch4, chunk4, 0)
        plsc.subcore_barrier()
        for k, sz in ((0, B), (1, RPT1 - B)):
            pltpu.sync_copy(acc.at[pl.ds(lo + k * B, sz)],
                            bufs.at[0].at[pl.ds(0, sz)])
            pltpu.sync_copy(
                bufs.at[0].at[pl.ds(0, sz)],
                out_hbm.at[cid, pl.ds(p * RPP + lo + k * B, sz)])
        plsc.subcore_barrier()


_agg1_kernel = functools.partial(
    pl.kernel,
    out_type=jax.ShapeDtypeStruct((NC, OUT1_ROWS, D), jnp.float32),
    mesh=_mesh,
    compiler_params=pltpu.CompilerParams(needs_layout_passes=False),
    scratch_types=[
        pltpu.VMEM((NCH, B), jnp.int32),
        pltpu.VMEM((NCHX, B), jnp.int32),
        pltpu.VMEM((LLEN,), jnp.int32),
        pltpu.VMEM((LLEN,), jnp.int32),
        pltpu.VMEM((LLEN // B, B), jnp.int32),
        pltpu.VMEM((NBUF, B, D), jnp.float32),
        pltpu.VMEM_SHARED((ACC1_ROWS, D), jnp.float32),
        pltpu.SemaphoreType.DMA,
        pltpu.SemaphoreType.DMA,
    ],
)(_agg1_body)


def _agg2_body(yt_hbm, srcs_hbm, dsts_hbm, out_hbm, src_v, dst_v, bufs, acc,
               sem):
    cid = lax.axis_index("c")
    sid = lax.axis_index("s")
    wid = cid * NS + sid

    pltpu.sync_copy(srcs_hbm.at[wid], src_v)
    pltpu.sync_copy(dsts_hbm.at[wid], dst_v)

    def zfill(i, _):
        for f in range(2):
            acc[f, pl.ds(i * 16, 16)] = jnp.zeros((16,), jnp.float32)
        return 0

    lax.fori_loop(0, NPAD // 16, zfill, 0)

    for f in range(2):
        yf = yt_hbm.at[pl.ds(f * NPAD, NPAD)]
        fidx = jnp.full((16,), f, jnp.int32)

        def chunk4(i, _):
            j0 = i * NBUF
            hs = []
            for k in range(NBUF):
                hs.append(pltpu.async_copy(
                    yf.at[src_v.at[j0 + k]], bufs.at[k], sem))
            for k in range(NBUF):
                hs[k].wait()
                for s in range(B // 16):
                    vals = bufs[k, pl.ds(s * 16, 16)]
                    idx = dst_v[j0 + k, pl.ds(s * 16, 16)]
                    plsc.addupdate_scatter(acc, [fidx, idx], vals)
            return 0

        lax.fori_loop(0, NCH // NBUF, chunk4, 0)
    pltpu.sync_copy(acc, out_hbm.at[wid])


_agg2_kernel = functools.partial(
    pl.kernel,
    out_type=jax.ShapeDtypeStruct((NW, 2, NPAD), jnp.float32),
    mesh=_mesh,
    compiler_params=pltpu.CompilerParams(needs_layout_passes=False, internal_scratch_in_bytes=0),
    scratch_types=[
        pltpu.VMEM((NCH, B), jnp.int32),
        pltpu.VMEM((NCH, B), jnp.int32),
        pltpu.VMEM((NBUF, B), jnp.float32),
        pltpu.VMEM((2, NPAD), jnp.float32),
        pltpu.SemaphoreType.DMA,
    ],
)(_agg2_body)


def _tc_a_body(x_ref, w_ref, dp_ref, y_ref):
    deg = jnp.sum(dp_ref[0], axis=0) + 1.0
    dis = lax.rsqrt(deg)
    xw = jnp.dot(x_ref[...], w_ref[...], preferred_element_type=jnp.float32)
    y_ref[...] = xw * dis[:, None]


def _tc_b_body(p_ref, y1_ref, dp_ref, b1_ref, w2_ref, y2t_ref):
    deg = jnp.sum(dp_ref[0], axis=0) + 1.0
    dis = lax.rsqrt(deg)
    agg = p_ref[0] + p_ref[1]
    h = dis[:, None] * (agg + y1_ref[...]) + b1_ref[...]
    h = jnp.maximum(h, 0.0)
    hw = jnp.dot(h, w2_ref[...], preferred_element_type=jnp.float32)
    y2t_ref[...] = (hw * dis[:, None]).T


def _tc_d_body(q_ref, y2t_ref, dp_ref, b2_ref, out_ref):
    deg = jnp.sum(dp_ref[...], axis=0) + 1.0
    dis = lax.rsqrt(deg)
    agg = jnp.sum(q_ref[...], axis=0)
    out_ref[...] = dis[None, :] * (agg + y2t_ref[...]) + b2_ref[...]


_RB = 1024
_G = NPAD // _RB


def _tc_a(xp, W1, dp):
    return pl.pallas_call(
        _tc_a_body,
        grid=(_G,),
        in_specs=[
            pl.BlockSpec((_RB, D), lambda i: (i, 0)),
            pl.BlockSpec((D, D), lambda i: (0, 0)),
            pl.BlockSpec((1, NW, _RB), lambda i: (i, 0, 0)),
        ],
        out_specs=pl.BlockSpec((_RB, D), lambda i: (i, 0)),
        out_shape=jax.ShapeDtypeStruct((NPAD, D), jnp.float32),
    )(xp, W1, dp)


def _tc_b(p, y1, dp, b1, W2):
    return pl.pallas_call(
        _tc_b_body,
        grid=(_G,),
        in_specs=[
            pl.BlockSpec((NC, _RB, D), lambda i: (0, i, 0)),
            pl.BlockSpec((_RB, D), lambda i: (i, 0)),
            pl.BlockSpec((1, NW, _RB), lambda i: (i, 0, 0)),
            pl.BlockSpec((1, D), lambda i: (0, 0)),
            pl.BlockSpec((D, 2), lambda i: (0, 0)),
        ],
        out_specs=pl.BlockSpec((2, _RB), lambda i: (0, i)),
        out_shape=jax.ShapeDtypeStruct((2, NPAD), jnp.float32),
    )(p, y1, dp, b1, W2)


def _tc_d(q, y2t, dp, b2):
    return pl.pallas_call(
        _tc_d_body,
        in_specs=[
            pl.BlockSpec((NW, 2, NPAD), lambda: (0, 0, 0)),
            pl.BlockSpec((2, NPAD), lambda: (0, 0)),
            pl.BlockSpec((NW, NPAD), lambda: (0, 0)),
            pl.BlockSpec((2, 1), lambda: (0, 0)),
        ],
        out_specs=pl.BlockSpec((2, NPAD), lambda: (0, 0)),
        out_shape=jax.ShapeDtypeStruct((2, NPAD), jnp.float32),
    )(q, y2t, dp, b2)


@jax.jit
def kernel(x, edge_index, W1, b1, W2, b2):
    ei = edge_index.astype(jnp.int32)
    srcf = jnp.concatenate([ei[0], jnp.zeros((E_PAD - E,), jnp.int32)])
    dstf = jnp.concatenate([ei[1], jnp.full((E_PAD - E,), GARBAGE, jnp.int32)])
    src = srcf.reshape(NW, NCH, B)
    dst = dstf.reshape(NW, NCH, B)
    s12 = srcf * 4096
    pk0 = (s12 + jnp.where(dstf < RPP, dstf, GARB_LOCAL)).reshape(
        NW, NCH, B)
    pk1 = (s12 + jnp.where((dstf >= RPP) & (dstf < 2 * RPP), dstf - RPP,
                           GARB_LOCAL)).reshape(NW, NCH, B)
    pk2 = (s12 + jnp.where(dstf >= 2 * RPP, dstf - 2 * RPP,
                           GARB_LOCAL)).reshape(NW, NCH, B)
    xp = jnp.pad(x, ((0, NPAD - N), (0, 0)))

    degp = _deg_kernel(dst)
    dpr = jnp.transpose(degp.reshape(NW, _G, _RB), (1, 0, 2))

    y1 = _tc_a(xp, W1, dpr)
    p = _agg1_kernel(y1, pk0, pk1, pk2)
    y2t = _tc_b(p, y1, dpr, b1.reshape(1, D), W2)
    q = _agg2_kernel(y2t.reshape(2 * NPAD), src, dst)
    outt = _tc_d(q, y2t, degp, b2.reshape(2, 1))
    return outt[:, :N].T


# speedup vs baseline: 9.9994x; 1.0548x over previous
import functools

import jax
import jax.numpy as jnp
from jax import lax
from jax.experimental import pallas as pl
from jax.experimental.pallas import tpu as pltpu
from jax.experimental.pallas import tpu_sc as plsc

N = 10000
D = 128
E = 320000

NC = 2
NS = 16
NW = NC * NS

B = 128
NCH = 80
E_PAD = NW * NCH * B
NPAD = 10240
GARBAGE = N + 100
NPASS = 3
RPP = 3456
ACC1_ROWS = RPP + 8
GARB_LOCAL = RPP
RPT1 = RPP // NS
OUT1_ROWS = NPASS * RPP
C0 = 56
NCHX = NCH - C0
LLEN = (NCH + NCHX + 4) * B
NBUF = 2

_mesh = plsc.VectorSubcoreMesh(core_axis_name="c", subcore_axis_name="s")


def _deg_body(dsts_hbm, degp_hbm, dst_v, acc, sem):
    cid = lax.axis_index("c")
    sid = lax.axis_index("s")
    wid = cid * NS + sid

    pltpu.sync_copy(dsts_hbm.at[wid], dst_v)

    def fill(i, _):
        acc[pl.ds(i * 16, 16)] = jnp.zeros((16,), jnp.float32)
        return 0

    lax.fori_loop(0, NPAD // 16, fill, 0)

    ones16 = jnp.ones((16,), jnp.float32)

    def chunk(j, _):
        for s in range(B // 16):
            idx = dst_v[j, pl.ds(s * 16, 16)]
            plsc.addupdate_scatter(acc, [idx], ones16)
        return 0

    lax.fori_loop(0, NCH, chunk, 0)
    pltpu.sync_copy(acc, degp_hbm.at[wid])


_deg_kernel = functools.partial(
    pl.kernel,
    out_type=jax.ShapeDtypeStruct((NW, NPAD), jnp.float32),
    mesh=_mesh,
    compiler_params=pltpu.CompilerParams(needs_layout_passes=False, internal_scratch_in_bytes=0),
    scratch_types=[
        pltpu.VMEM((NCH, B), jnp.int32),
        pltpu.VMEM((NPAD,), jnp.float32),
        pltpu.SemaphoreType.DMA,
    ],
)(_deg_body)


def _agg1_body(y_hbm, p0_hbm, p1_hbm, p2_hbm, out_hbm, pk_v, pkx_v, lsrc,
               ldst, ldst2, bufs, acc, sem, sems):
    cid = lax.axis_index("c")
    sid = lax.axis_index("s")
    wid = cid * NS + sid

    n_main = jnp.where(cid == 1, C0 * (B // 16), NCH * (B // 16))
    n_x = jnp.where(cid == 1, 0, NCHX * (B // 16))

    lo = sid * RPT1
    for p, p_hbm in ((0, p0_hbm), (1, p1_hbm), (2, p2_hbm)):
        pltpu.sync_copy(p_hbm.at[wid], pk_v)

        @pl.when(cid == 0)
        def _():
            pltpu.sync_copy(p_hbm.at[wid + NS].at[pl.ds(C0, NCHX)], pkx_v)

        def zfill(i, _):
            bufs[0, lax.div(i, 8), pl.ds(lax.rem(i, 8) * 16, 16)] = (
                jnp.zeros((16,), jnp.float32))
            return 0

        lax.fori_loop(0, B * 8, zfill, 0)
        for k, sz in ((0, B), (1, RPT1 - B)):
            pltpu.sync_copy(bufs.at[0].at[pl.ds(0, sz)],
                            acc.at[pl.ds(lo + k * B, sz)])

        def pfill(i, _):
            lsrc[pl.ds(i * 16, 16)] = jnp.zeros((16,), jnp.int32)
            ldst[pl.ds(i * 16, 16)] = jnp.full((16,), GARB_LOCAL, jnp.int32)
            return 0

        lax.fori_loop(0, LLEN // 16, pfill, 0)

        def comp(t, cnt):
            r = lax.div(t, B // 16)
            c = lax.rem(t, B // 16)
            v = pk_v[r, pl.ds(c * 16, 16)]
            vs = lax.shift_right_logical(v, 12)
            vd = lax.bitwise_and(v, 4095)
            m = vd < GARB_LOCAL
            plsc.store_compressed(lsrc.at[pl.ds(cnt, 16)], vs, mask=m)
            plsc.store_compressed(ldst.at[pl.ds(cnt, 16)], vd, mask=m)
            return cnt + jnp.max(plsc.all_reduce_population_count(m))

        cnt = lax.fori_loop(0, n_main, comp, jnp.int32(0))

        def compx(t, cnt):
            r = lax.div(t, B // 16)
            c = lax.rem(t, B // 16)
            v = pkx_v[r, pl.ds(c * 16, 16)]
            vs = lax.shift_right_logical(v, 12)
            vd = lax.bitwise_and(v, 4095)
            m = vd < GARB_LOCAL
            plsc.store_compressed(lsrc.at[pl.ds(cnt, 16)], vs, mask=m)
            plsc.store_compressed(ldst.at[pl.ds(cnt, 16)], vd, mask=m)
            return cnt + jnp.max(plsc.all_reduce_population_count(m))

        cnt = lax.fori_loop(0, n_x, compx, cnt)
        nch4 = lax.div(cnt + (NBUF * B - 1), NBUF * B)

        def lcopy(j, _):
            for st in range(B // 16):
                ldst2[j, pl.ds(st * 16, 16)] = ldst[pl.ds(j * B + st * 16, 16)]
            return 0

        lax.fori_loop(0, nch4 * NBUF, lcopy, 0)
        plsc.subcore_barrier()

        def chunk4(i, _):
            j0 = i * NBUF
            hs = []
            for k in range(NBUF):
                hs.append(pltpu.async_copy(
                    y_hbm.at[lsrc.at[pl.ds((j0 + k) * B, B)]], bufs.at[k],
                    sem))
            ss = []
            for k in range(NBUF):
                hs[k].wait()
                ss.append(pltpu.async_copy(
                    bufs.at[k], acc.at[ldst2.at[j0 + k]], sems, add=True))
            for k in range(NBUF):
                ss[k].wait()
            return 0

        lax.fori_loop(0, nch4, chunk4, 0)
        plsc.subcore_barrier()
        for k, sz in ((0, B), (1, RPT1 - B)):
            pltpu.sync_copy(acc.at[pl.ds(lo + k * B, sz)],
                            bufs.at[0].at[pl.ds(0, sz)])
            pltpu.sync_copy(
                bufs.at[0].at[pl.ds(0, sz)],
                out_hbm.at[cid, pl.ds(p * RPP + lo + k * B, sz)])
        plsc.subcore_barrier()


_agg1_kernel = functools.partial(
    pl.kernel,
    out_type=jax.ShapeDtypeStruct((NC, OUT1_ROWS, D), jnp.float32),
    mesh=_mesh,
    compiler_params=pltpu.CompilerParams(needs_layout_passes=False),
    scratch_types=[
        pltpu.VMEM((NCH, B), jnp.int32),
        pltpu.VMEM((NCHX, B), jnp.int32),
        pltpu.VMEM((LLEN,), jnp.int32),
        pltpu.VMEM((LLEN,), jnp.int32),
        pltpu.VMEM((LLEN // B, B), jnp.int32),
        pltpu.VMEM((NBUF, B, D), jnp.float32),
        pltpu.VMEM_SHARED((ACC1_ROWS, D), jnp.float32),
        pltpu.SemaphoreType.DMA,
        pltpu.SemaphoreType.DMA,
    ],
)(_agg1_body)


def _agg2_body(yt_hbm, srcs_hbm, dsts_hbm, out_hbm, src_v, dst_v, bufs, acc,
               sem):
    cid = lax.axis_index("c")
    sid = lax.axis_index("s")
    wid = cid * NS + sid

    pltpu.sync_copy(srcs_hbm.at[wid], src_v)
    pltpu.sync_copy(dsts_hbm.at[wid], dst_v)

    def zfill(i, _):
        for f in range(2):
            acc[f, pl.ds(i * 16, 16)] = jnp.zeros((16,), jnp.float32)
        return 0

    lax.fori_loop(0, NPAD // 16, zfill, 0)

    fidx = [jnp.full((16,), f, jnp.int32) for f in range(2)]
    yfs = [yt_hbm.at[pl.ds(f * NPAD, NPAD)] for f in range(2)]

    def chunk2(i, _):
        hs = []
        for k in range(2):
            j = i * 2 + k
            for f in range(2):
                hs.append(pltpu.async_copy(
                    yfs[f].at[src_v.at[j]], bufs.at[k * 2 + f], sem))
        for k in range(2):
            j = i * 2 + k
            for f in range(2):
                hs[k * 2 + f].wait()
                for st in range(B // 16):
                    vals = bufs[k * 2 + f, pl.ds(st * 16, 16)]
                    idx = dst_v[j, pl.ds(st * 16, 16)]
                    plsc.addupdate_scatter(acc, [fidx[f], idx], vals)
        return 0

    lax.fori_loop(0, NCH // 2, chunk2, 0)
    pltpu.sync_copy(acc, out_hbm.at[wid])


_agg2_kernel = functools.partial(
    pl.kernel,
    out_type=jax.ShapeDtypeStruct((NW, 2, NPAD), jnp.float32),
    mesh=_mesh,
    compiler_params=pltpu.CompilerParams(needs_layout_passes=False, internal_scratch_in_bytes=0),
    scratch_types=[
        pltpu.VMEM((NCH, B), jnp.int32),
        pltpu.VMEM((NCH, B), jnp.int32),
        pltpu.VMEM((4, B), jnp.float32),
        pltpu.VMEM((2, NPAD), jnp.float32),
        pltpu.SemaphoreType.DMA,
    ],
)(_agg2_body)


def _tc_a_body(x_ref, w_ref, dp_ref, y_ref):
    deg = jnp.sum(dp_ref[0], axis=0) + 1.0
    dis = lax.rsqrt(deg)
    xw = jnp.dot(x_ref[...], w_ref[...], preferred_element_type=jnp.float32)
    y_ref[...] = xw * dis[:, None]


def _tc_b_body(p_ref, y1_ref, dp_ref, b1_ref, w2_ref, y2t_ref):
    deg = jnp.sum(dp_ref[0], axis=0) + 1.0
    dis = lax.rsqrt(deg)
    agg = p_ref[0] + p_ref[1]
    h = dis[:, None] * (agg + y1_ref[...]) + b1_ref[...]
    h = jnp.maximum(h, 0.0)
    hw = jnp.dot(h, w2_ref[...], preferred_element_type=jnp.float32)
    y2t_ref[...] = (hw * dis[:, None]).T


def _tc_d_body(q_ref, y2t_ref, dp_ref, b2_ref, out_ref):
    deg = jnp.sum(dp_ref[...], axis=0) + 1.0
    dis = lax.rsqrt(deg)
    agg = jnp.sum(q_ref[...], axis=0)
    out_ref[...] = dis[None, :] * (agg + y2t_ref[...]) + b2_ref[...]


_RB = 1024
_G = NPAD // _RB


def _tc_a(xp, W1, dp):
    return pl.pallas_call(
        _tc_a_body,
        grid=(_G,),
        in_specs=[
            pl.BlockSpec((_RB, D), lambda i: (i, 0)),
            pl.BlockSpec((D, D), lambda i: (0, 0)),
            pl.BlockSpec((1, NW, _RB), lambda i: (i, 0, 0)),
        ],
        out_specs=pl.BlockSpec((_RB, D), lambda i: (i, 0)),
        out_shape=jax.ShapeDtypeStruct((NPAD, D), jnp.float32),
    )(xp, W1, dp)


def _tc_b(p, y1, dp, b1, W2):
    return pl.pallas_call(
        _tc_b_body,
        grid=(_G,),
        in_specs=[
            pl.BlockSpec((NC, _RB, D), lambda i: (0, i, 0)),
            pl.BlockSpec((_RB, D), lambda i: (i, 0)),
            pl.BlockSpec((1, NW, _RB), lambda i: (i, 0, 0)),
            pl.BlockSpec((1, D), lambda i: (0, 0)),
            pl.BlockSpec((D, 2), lambda i: (0, 0)),
        ],
        out_specs=pl.BlockSpec((2, _RB), lambda i: (0, i)),
        out_shape=jax.ShapeDtypeStruct((2, NPAD), jnp.float32),
    )(p, y1, dp, b1, W2)


def _tc_d(q, y2t, dp, b2):
    return pl.pallas_call(
        _tc_d_body,
        in_specs=[
            pl.BlockSpec((NW, 2, NPAD), lambda: (0, 0, 0)),
            pl.BlockSpec((2, NPAD), lambda: (0, 0)),
            pl.BlockSpec((NW, NPAD), lambda: (0, 0)),
            pl.BlockSpec((2, 1), lambda: (0, 0)),
        ],
        out_specs=pl.BlockSpec((2, NPAD), lambda: (0, 0)),
        out_shape=jax.ShapeDtypeStruct((2, NPAD), jnp.float32),
    )(q, y2t, dp, b2)


@jax.jit
def kernel(x, edge_index, W1, b1, W2, b2):
    ei = edge_index.astype(jnp.int32)
    srcf = jnp.concatenate([ei[0], jnp.zeros((E_PAD - E,), jnp.int32)])
    dstf = jnp.concatenate([ei[1], jnp.full((E_PAD - E,), GARBAGE, jnp.int32)])
    src = srcf.reshape(NW, NCH, B)
    dst = dstf.reshape(NW, NCH, B)
    s12 = srcf * 4096
    pk0 = (s12 + jnp.where(dstf < RPP, dstf, GARB_LOCAL)).reshape(
        NW, NCH, B)
    pk1 = (s12 + jnp.where((dstf >= RPP) & (dstf < 2 * RPP), dstf - RPP,
                           GARB_LOCAL)).reshape(NW, NCH, B)
    pk2 = (s12 + jnp.where(dstf >= 2 * RPP, dstf - 2 * RPP,
                           GARB_LOCAL)).reshape(NW, NCH, B)
    xp = jnp.pad(x, ((0, NPAD - N), (0, 0)))

    degp = _deg_kernel(dst)
    dpr = jnp.transpose(degp.reshape(NW, _G, _RB), (1, 0, 2))

    y1 = _tc_a(xp, W1, dpr)
    p = _agg1_kernel(y1, pk0, pk1, pk2)
    y2t = _tc_b(p, y1, dpr, b1.reshape(1, D), W2)
    q = _agg2_kernel(y2t.reshape(2 * NPAD), src, dst)
    outt = _tc_d(q, y2t, degp, b2.reshape(2, 1))
    return outt[:, :N].T
